# Initial kernel scaffold; baseline (speedup 1.0000x reference)
import jax
import jax.numpy as jnp
from jax.experimental import pallas as pl


def kernel(tensor):
    raise NotImplementedError("write your pallas kernel here")


# speedup vs baseline: 5.2863x; 5.2863x over previous
import functools

import jax
import jax.numpy as jnp
from jax import lax
from jax.experimental import pallas as pl
from jax.experimental.pallas import tpu as pltpu
from jax.experimental.pallas import tpu_sc as plsc

_B, _H, _W, _C = 4, 384, 384, 96
_F = 16
_HO = (_H - _F) // _F + 1
_WO = (_W - _F) // _F + 1
_U = _B * _HO
_ROW = _F * _C
_C_OUT = _F * _F * _C

_NW = 32
_UPW = _U // _NW


def _fold_body(x_hbm, out_hbm, buf, sem):
    wid = lax.axis_index("s") * 2 + lax.axis_index("c")

    def fx_body(fx, _):
        for du in range(_UPW):
            u = wid * _UPW + du
            pltpu.async_copy(x_hbm.at[u, :, fx, :], buf, sem).wait()
            pltpu.async_copy(buf, out_hbm.at[u, fx], sem).wait()
        return ()

    lax.fori_loop(0, _WO, fx_body, ())


_fold = functools.partial(
    pl.kernel,
    mesh=plsc.VectorSubcoreMesh(core_axis_name="c", subcore_axis_name="s"),
    out_type=jax.ShapeDtypeStruct((_U, _WO, _F, _ROW), jnp.float32),
    scratch_types=[
        pltpu.VMEM((_F, _ROW), jnp.float32),
        pltpu.SemaphoreType.DMA,
    ],
)(_fold_body)


def kernel(tensor):
    x = tensor.reshape(_U, _F, _WO, _ROW)
    out = _fold(x)
    return out.reshape(_B, _WO, _HO, _C_OUT)


# speedup vs baseline: 5.5662x; 1.0530x over previous
import functools

import jax
import jax.numpy as jnp
from jax import lax
from jax.experimental import pallas as pl
from jax.experimental.pallas import tpu as pltpu
from jax.experimental.pallas import tpu_sc as plsc

_B, _H, _W, _C = 4, 384, 384, 96
_F = 16
_HO = (_H - _F) // _F + 1
_WO = (_W - _F) // _F + 1
_U = _B * _HO
_ROW = _F * _C
_C_OUT = _F * _F * _C

_NW = 32
_UPW = _U // _NW
_UNITS = _UPW * _WO
_GRP = 2
_NSET = 2
_ITERS = _UNITS // (_GRP * _NSET)


def _fold_body(x_hbm, out_hbm, bufs, in_a, in_b, out_a, out_b):
    wid = lax.axis_index("s") * 2 + lax.axis_index("c")
    in_sems = (in_a, in_b)
    out_sems = (out_a, out_b)

    def unit_idx(t):
        du = t // _WO
        fx = t % _WO
        return wid * _UPW + du, fx

    def drain(sem, k):
        pltpu.make_async_copy(x_hbm.at[0, :, 0, :], bufs.at[k], sem).wait()

    def iter_body(i, _):
        for s in range(_NSET):
            g = i * _NSET + s

            @pl.when(g >= _NSET)
            def _drain_prev_writes():
                for j in range(_GRP):
                    drain(out_sems[s], s * _GRP + j)

            handles = []
            for j in range(_GRP):
                u, fx = unit_idx(g * _GRP + j)
                handles.append(pltpu.async_copy(
                    x_hbm.at[u, :, fx, :], bufs.at[s * _GRP + j], in_sems[s]))
            for h in handles:
                h.wait()
            for j in range(_GRP):
                u, fx = unit_idx(g * _GRP + j)
                pltpu.async_copy(
                    bufs.at[s * _GRP + j], out_hbm.at[u, fx], out_sems[s])
        return ()

    lax.fori_loop(0, _ITERS, iter_body, ())
    for s in range(_NSET):
        for j in range(_GRP):
            drain(out_sems[s], s * _GRP + j)


_fold = functools.partial(
    pl.kernel,
    mesh=plsc.VectorSubcoreMesh(core_axis_name="c", subcore_axis_name="s"),
    out_type=jax.ShapeDtypeStruct((_U, _WO, _F, _ROW), jnp.float32),
    scratch_types=[
        pltpu.VMEM((_NSET * _GRP, _F, _ROW), jnp.float32),
        pltpu.SemaphoreType.DMA,
        pltpu.SemaphoreType.DMA,
        pltpu.SemaphoreType.DMA,
        pltpu.SemaphoreType.DMA,
    ],
)(_fold_body)


def kernel(tensor):
    x = tensor.reshape(_U, _F, _WO, _ROW)
    out = _fold(x)
    return out.reshape(_B, _WO, _HO, _C_OUT)


# speedup vs baseline: 7.0972x; 1.2751x over previous
import functools

import jax
import jax.numpy as jnp
from jax import lax
from jax.experimental import pallas as pl
from jax.experimental.pallas import tpu as pltpu
from jax.experimental.pallas import tpu_sc as plsc

_B, _H, _W, _C = 4, 384, 384, 96
_F = 16
_HO = (_H - _F) // _F + 1
_WO = (_W - _F) // _F + 1
_U = _B * _HO
_ROW = _F * _C
_C_OUT = _F * _F * _C
_L = 16

_NW = 32
_UPW = _U // _NW
_UNITS = _UPW * _WO
_NS = 2


def _fold_body(x_hbm, out_hbm, buf_a, buf_b, *sems):
    in_sems = sems[:_NS]
    out_sems = sems[_NS:]
    wid = lax.axis_index("s") * 2 + lax.axis_index("c")

    def unit_idx(t):
        du = t // _WO
        fx = t % _WO
        u = wid * _UPW + du
        b = u // _HO
        i = u % _HO
        return b, i, fx

    def fire_gather(g, s):
        b, i, fx = unit_idx(g)
        pltpu.async_copy(
            x_hbm.at[b, pl.ds(i * _F, _F), pl.ds(fx * _F, _F), :],
            buf_a.at[s], in_sems[s])

    def wait_gather(s):
        pltpu.make_async_copy(
            x_hbm.at[0, pl.ds(0, _F), pl.ds(0, _F), :],
            buf_a.at[s], in_sems[s]).wait()

    def repack(s):
        def row(r, _):
            yi = r // _F
            xi = r % _F
            base = r * _C
            for c in range(_C // _L):
                buf_b[s, pl.ds(base + c * _L, _L)] = (
                    buf_a[s, yi, xi, pl.ds(c * _L, _L)])
            return ()
        lax.fori_loop(0, _F * _F, row, (), unroll=4)

    def fire_writes(g, s):
        b, i, fx = unit_idx(g)
        for yi in range(_F):
            pltpu.async_copy(
                buf_b.at[s, pl.ds(yi * _ROW, _ROW)],
                out_hbm.at[b, i, fx, pl.ds(yi * _ROW, _ROW)], out_sems[s])

    def wait_writes(s):
        for _ in range(_F):
            pltpu.make_async_copy(
                buf_b.at[s, pl.ds(0, _ROW)],
                out_hbm.at[0, 0, 0, pl.ds(0, _ROW)], out_sems[s]).wait()

    fire_gather(jnp.int32(0), 0)

    def iter_body(t, _):
        for s in range(_NS):
            g = t * _NS + s

            @pl.when(g + 1 < _UNITS)
            def _prefetch():
                fire_gather(g + 1, 1 - s)

            wait_gather(s)

            @pl.when(g >= _NS)
            def _recycle():
                wait_writes(s)

            repack(s)
            fire_writes(g, s)
        return ()

    lax.fori_loop(0, _UNITS // _NS, iter_body, ())
    for s in range(_NS):
        wait_writes(s)


_fold = functools.partial(
    pl.kernel,
    mesh=plsc.VectorSubcoreMesh(core_axis_name="c", subcore_axis_name="s"),
    out_type=jax.ShapeDtypeStruct((_B, _WO, _HO, _C_OUT), jnp.float32),
    scratch_types=(
        [pltpu.VMEM((_NS, _F, _F, _C), jnp.float32),
         pltpu.VMEM((_NS, _C_OUT), jnp.float32)]
        + [pltpu.SemaphoreType.DMA] * (2 * _NS)
    ),
)(_fold_body)


def kernel(tensor):
    return _fold(tensor)
